# baseline (speedup 1.0000x reference)
import functools

import jax
import jax.numpy as jnp
from jax import lax
from jax.experimental import pallas as pl
from jax.experimental.pallas import tpu as pltpu
from jax.experimental.pallas import tpu_sc as plsc

N_NODES = 10000
D = 128
NC = 2
NS = 16
NW = NC * NS
CHUNK = 64
N_CHUNKS = 160
N_PHASES = 5
PH_CHUNKS = N_CHUNKS // N_PHASES
E_PAD = NW * N_CHUNKS * CHUNK
N_ACC = 10240
ROWS_PER_TILE = N_ACC // NS
ZROWS = CHUNK


def _sc_scatter(X, src, dst, ew):
    mesh = plsc.VectorSubcoreMesh(
        core_axis_name="c", subcore_axis_name="s",
        num_cores=NC, num_subcores=NS)

    @functools.partial(
        pl.kernel,
        out_type=jax.ShapeDtypeStruct((NC, N_ACC, D), jnp.float32),
        mesh=mesh,
        scratch_types=[
            pltpu.VMEM((PH_CHUNKS, CHUNK), jnp.int32),
            pltpu.VMEM((PH_CHUNKS, CHUNK), jnp.int32),
            pltpu.VMEM((PH_CHUNKS, CHUNK), jnp.float32),
            pltpu.VMEM((CHUNK, D), jnp.float32),
            pltpu.VMEM((CHUNK, D), jnp.float32),
            pltpu.VMEM((CHUNK, D), jnp.float32),
            pltpu.VMEM((CHUNK, D), jnp.float32),
            pltpu.VMEM_SHARED((N_ACC, D), jnp.float32),
            pltpu.SemaphoreType.DMA,
            pltpu.SemaphoreType.DMA,
            pltpu.SemaphoreType.DMA,
            pltpu.SemaphoreType.DMA,
        ],
        compiler_params=pltpu.CompilerParams(use_tc_tiling_on_sc=False),
    )
    def k(x_hbm, src_hbm, dst_hbm, ew_hbm, out_hbm,
          src_v, dst_v, ew_v, g0, g1, s0, s1, acc,
          sem_g0, sem_g1, sem_s0, sem_s1):
        gbufs = (g0, g1)
        sbufs = (s0, s1)
        sems_g = (sem_g0, sem_g1)
        sems_s = (sem_s0, sem_s1)
        c = lax.axis_index("c")
        s = lax.axis_index("s")
        gwid = c * NS + s

        def zrow(i, carry):
            for v in range(D // 16):
                g0[i, pl.ds(16 * v, 16)] = jnp.zeros((16,), jnp.float32)
            return carry
        lax.fori_loop(0, ZROWS, zrow, 0)
        base = s * ROWS_PER_TILE
        for t in range(ROWS_PER_TILE // ZROWS):
            pltpu.sync_copy(g0, acc.at[pl.ds(base + t * ZROWS, ZROWS)])
        plsc.subcore_barrier()

        def scale(j, src_buf, dst_buf):
            def group(g, gcarry):
                wv = ew_v[j, pl.ds(g * 16, 16)]
                ws = [jnp.full((16,), wv[i], jnp.float32) for i in range(16)]

                def col(v, ccarry):
                    sl = pl.ds(v * 16, 16)
                    for i in range(16):
                        e = g * 16 + i
                        dst_buf[e, sl] = src_buf[e, sl] * ws[i]
                    return ccarry
                lax.fori_loop(0, D // 16, col, 0)
                return gcarry
            lax.fori_loop(0, CHUNK // 16, group, 0)

        for phase in range(N_PHASES):
            p0 = phase * PH_CHUNKS
            pltpu.sync_copy(src_hbm.at[gwid, pl.ds(p0, PH_CHUNKS)], src_v)
            pltpu.sync_copy(dst_hbm.at[gwid, pl.ds(p0, PH_CHUNKS)], dst_v)
            pltpu.sync_copy(ew_hbm.at[gwid, pl.ds(p0, PH_CHUNKS)], ew_v)

            for b in range(2):
                pltpu.async_copy(x_hbm.at[src_v.at[b]], gbufs[b], sems_g[b])

            def pair(q, carry):
                for b in range(2):
                    j = 2 * q + b
                    jn = jnp.minimum(j + 2, PH_CHUNKS - 1)

                    pltpu.make_async_copy(
                        x_hbm.at[src_v.at[j]], gbufs[b], sems_g[b]).wait()

                    @pl.when(j >= 2)
                    def _():
                        pltpu.make_async_copy(
                            sbufs[b], acc.at[dst_v.at[j]], sems_s[b]).wait()
                    scale(j, gbufs[b], sbufs[b])
                    pltpu.async_copy(sbufs[b], acc.at[dst_v.at[j]],
                                     sems_s[b], add=True)
                    pltpu.async_copy(x_hbm.at[src_v.at[jn]], gbufs[b],
                                     sems_g[b])
                return carry
            lax.fori_loop(0, PH_CHUNKS // 2, pair, 0)
            for b in range(2):
                pltpu.make_async_copy(
                    x_hbm.at[src_v.at[0]], gbufs[b], sems_g[b]).wait()
                pltpu.make_async_copy(
                    sbufs[b], acc.at[dst_v.at[0]], sems_s[b]).wait()

        plsc.subcore_barrier()
        for t in range(ROWS_PER_TILE // ZROWS):
            lo = base + t * ZROWS
            pltpu.sync_copy(acc.at[pl.ds(lo, ZROWS)],
                            out_hbm.at[c, pl.ds(lo, ZROWS)])

    return k(X, src, dst, ew)


def _tc_body(p0_ref, p1_ref, w_ref, b_ref, o_ref):
    o_ref[...] = (
        lax.dot_general(p0_ref[...] + p1_ref[...], w_ref[...],
                        (((1,), (1,)), ((), ())),
                        preferred_element_type=jnp.float32)
        + b_ref[...])


def _tc_linear(p0, p1, W, b2d):
    rows = 1000
    return pl.pallas_call(
        _tc_body,
        grid=(N_NODES // rows,),
        in_specs=[
            pl.BlockSpec((rows, D), lambda i: (i, 0)),
            pl.BlockSpec((rows, D), lambda i: (i, 0)),
            pl.BlockSpec((D, D), lambda i: (0, 0)),
            pl.BlockSpec((1, D), lambda i: (0, 0)),
        ],
        out_specs=pl.BlockSpec((rows, D), lambda i: (i, 0)),
        out_shape=jax.ShapeDtypeStruct((N_NODES, D), jnp.float32),
    )(p0, p1, W, b2d)


def kernel(X, edge_index, edge_weight, W, b):
    src = edge_index[1].astype(jnp.int32)
    dst = edge_index[0].astype(jnp.int32)
    ew = edge_weight.astype(jnp.float32)
    pad = E_PAD - src.shape[0]
    src = jnp.pad(src, (0, pad)).reshape(NW, N_CHUNKS, CHUNK)
    dst = jnp.pad(dst, (0, pad)).reshape(NW, N_CHUNKS, CHUNK)
    ew = jnp.pad(ew, (0, pad)).reshape(NW, N_CHUNKS, CHUNK)
    part = _sc_scatter(X, src, dst, ew)
    return _tc_linear(part[0, :N_NODES], part[1, :N_NODES], W,
                      b.reshape(1, D))


# speedup vs baseline: 1.0003x; 1.0003x over previous
import functools

import jax
import jax.numpy as jnp
from jax import lax
from jax.experimental import pallas as pl
from jax.experimental.pallas import tpu as pltpu
from jax.experimental.pallas import tpu_sc as plsc

N_NODES = 10000
D = 128
NC = 2
NS = 16
NW = NC * NS
CHUNK = 64
N_CHUNKS = 160
N_PHASES = 5
PH_CHUNKS = N_CHUNKS // N_PHASES
E_PAD = NW * N_CHUNKS * CHUNK
N_ACC = 10240
ROWS_PER_TILE = N_ACC // NS
ZROWS = CHUNK


def _sc_scatter(X, src, dst, ew):
    mesh = plsc.VectorSubcoreMesh(
        core_axis_name="c", subcore_axis_name="s",
        num_cores=NC, num_subcores=NS)

    @functools.partial(
        pl.kernel,
        out_type=jax.ShapeDtypeStruct((NC, N_ACC, D), jnp.float32),
        mesh=mesh,
        scratch_types=[
            pltpu.VMEM((PH_CHUNKS, CHUNK), jnp.int32),
            pltpu.VMEM((PH_CHUNKS, CHUNK), jnp.int32),
            pltpu.VMEM((PH_CHUNKS, CHUNK), jnp.float32),
            pltpu.VMEM((CHUNK, D), jnp.float32),
            pltpu.VMEM((CHUNK, D), jnp.float32),
            pltpu.VMEM((CHUNK, D), jnp.float32),
            pltpu.VMEM((CHUNK, D), jnp.float32),
            pltpu.VMEM_SHARED((N_ACC, D), jnp.float32),
            pltpu.SemaphoreType.DMA,
            pltpu.SemaphoreType.DMA,
            pltpu.SemaphoreType.DMA,
            pltpu.SemaphoreType.DMA,
        ],
        compiler_params=pltpu.CompilerParams(use_tc_tiling_on_sc=False),
    )
    def k(x_hbm, src_hbm, dst_hbm, ew_hbm, out_hbm,
          src_v, dst_v, ew_v, g0, g1, s0, s1, acc,
          sem_g0, sem_g1, sem_s0, sem_s1):
        gbufs = (g0, g1)
        sbufs = (s0, s1)
        sems_g = (sem_g0, sem_g1)
        sems_s = (sem_s0, sem_s1)
        c = lax.axis_index("c")
        s = lax.axis_index("s")
        gwid = c * NS + s

        def zrow(i, carry):
            for v in range(D // 16):
                g0[i, pl.ds(16 * v, 16)] = jnp.zeros((16,), jnp.float32)
            return carry
        lax.fori_loop(0, ZROWS, zrow, 0)
        base = s * ROWS_PER_TILE
        for t in range(ROWS_PER_TILE // ZROWS):
            pltpu.sync_copy(g0, acc.at[pl.ds(base + t * ZROWS, ZROWS)])
        plsc.subcore_barrier()

        def scale(j, src_buf, dst_buf):
            def group(g, gcarry):
                wv = ew_v[j, pl.ds(g * 16, 16)]
                ws = [jnp.full((16,), wv[i], jnp.float32) for i in range(16)]

                def col(v, ccarry):
                    sl = pl.ds(v * 16, 16)
                    for i in range(16):
                        e = g * 16 + i
                        dst_buf[e, sl] = src_buf[e, sl] * ws[i]
                    return ccarry
                lax.fori_loop(0, D // 16, col, 0)
                return gcarry
            lax.fori_loop(0, CHUNK // 16, group, 0)

        for phase in range(N_PHASES):
            p0 = phase * PH_CHUNKS
            pltpu.sync_copy(src_hbm.at[gwid, pl.ds(p0, PH_CHUNKS)], src_v)
            pltpu.sync_copy(dst_hbm.at[gwid, pl.ds(p0, PH_CHUNKS)], dst_v)
            pltpu.sync_copy(ew_hbm.at[gwid, pl.ds(p0, PH_CHUNKS)], ew_v)

            for b in range(2):
                pltpu.async_copy(x_hbm.at[src_v.at[b]], gbufs[b], sems_g[b])

            def pair(q, carry):
                for b in range(2):
                    j = 2 * q + b
                    jn = jnp.minimum(j + 2, PH_CHUNKS - 1)

                    pltpu.make_async_copy(
                        x_hbm.at[src_v.at[j]], gbufs[b], sems_g[b]).wait()

                    scale(j, gbufs[b], sbufs[b])
                    pltpu.async_copy(x_hbm.at[src_v.at[jn]], gbufs[b],
                                     sems_g[b])
                return carry
            lax.fori_loop(0, PH_CHUNKS // 2, pair, 0)
            for b in range(2):
                pltpu.make_async_copy(
                    x_hbm.at[src_v.at[0]], gbufs[b], sems_g[b]).wait()

        plsc.subcore_barrier()
        for t in range(ROWS_PER_TILE // ZROWS):
            lo = base + t * ZROWS
            pltpu.sync_copy(acc.at[pl.ds(lo, ZROWS)],
                            out_hbm.at[c, pl.ds(lo, ZROWS)])

    return k(X, src, dst, ew)


def _tc_body(p0_ref, p1_ref, w_ref, b_ref, o_ref):
    o_ref[...] = (
        lax.dot_general(p0_ref[...] + p1_ref[...], w_ref[...],
                        (((1,), (1,)), ((), ())),
                        preferred_element_type=jnp.float32)
        + b_ref[...])


def _tc_linear(p0, p1, W, b2d):
    rows = 1000
    return pl.pallas_call(
        _tc_body,
        grid=(N_NODES // rows,),
        in_specs=[
            pl.BlockSpec((rows, D), lambda i: (i, 0)),
            pl.BlockSpec((rows, D), lambda i: (i, 0)),
            pl.BlockSpec((D, D), lambda i: (0, 0)),
            pl.BlockSpec((1, D), lambda i: (0, 0)),
        ],
        out_specs=pl.BlockSpec((rows, D), lambda i: (i, 0)),
        out_shape=jax.ShapeDtypeStruct((N_NODES, D), jnp.float32),
    )(p0, p1, W, b2d)


def kernel(X, edge_index, edge_weight, W, b):
    src = edge_index[1].astype(jnp.int32)
    dst = edge_index[0].astype(jnp.int32)
    ew = edge_weight.astype(jnp.float32)
    pad = E_PAD - src.shape[0]
    src = jnp.pad(src, (0, pad)).reshape(NW, N_CHUNKS, CHUNK)
    dst = jnp.pad(dst, (0, pad)).reshape(NW, N_CHUNKS, CHUNK)
    ew = jnp.pad(ew, (0, pad)).reshape(NW, N_CHUNKS, CHUNK)
    part = _sc_scatter(X, src, dst, ew)
    return _tc_linear(part[0, :N_NODES], part[1, :N_NODES], W,
                      b.reshape(1, D))


# speedup vs baseline: 1.2125x; 1.2121x over previous
import functools

import jax
import jax.numpy as jnp
from jax import lax
from jax.experimental import pallas as pl
from jax.experimental.pallas import tpu as pltpu
from jax.experimental.pallas import tpu_sc as plsc

N_NODES = 10000
D = 128
NC = 2
NS = 16
NW = NC * NS
CHUNK = 64
N_CHUNKS = 160
N_PHASES = 5
PH_CHUNKS = N_CHUNKS // N_PHASES
E_PAD = NW * N_CHUNKS * CHUNK
N_ACC = 10240
ROWS_PER_TILE = N_ACC // NS
ZROWS = CHUNK


def _sc_scatter(X, src, dst, ew):
    mesh = plsc.VectorSubcoreMesh(
        core_axis_name="c", subcore_axis_name="s",
        num_cores=NC, num_subcores=NS)

    @functools.partial(
        pl.kernel,
        out_type=jax.ShapeDtypeStruct((NC, N_ACC, D), jnp.float32),
        mesh=mesh,
        scratch_types=[
            pltpu.VMEM((PH_CHUNKS, CHUNK), jnp.int32),
            pltpu.VMEM((PH_CHUNKS, CHUNK), jnp.int32),
            pltpu.VMEM((PH_CHUNKS, CHUNK), jnp.float32),
            pltpu.VMEM((CHUNK, D), jnp.float32),
            pltpu.VMEM((CHUNK, D), jnp.float32),
            pltpu.VMEM((CHUNK, D), jnp.float32),
            pltpu.VMEM((CHUNK, D), jnp.float32),
            pltpu.VMEM_SHARED((N_ACC, D), jnp.float32),
            pltpu.SemaphoreType.DMA,
            pltpu.SemaphoreType.DMA,
            pltpu.SemaphoreType.DMA,
            pltpu.SemaphoreType.DMA,
        ],
        compiler_params=pltpu.CompilerParams(use_tc_tiling_on_sc=False),
    )
    def k(x_hbm, src_hbm, dst_hbm, ew_hbm, out_hbm,
          src_v, dst_v, ew_v, g0, g1, s0, s1, acc,
          sem_g0, sem_g1, sem_s0, sem_s1):
        gbufs = (g0, g1)
        sbufs = (s0, s1)
        sems_g = (sem_g0, sem_g1)
        sems_s = (sem_s0, sem_s1)
        c = lax.axis_index("c")
        s = lax.axis_index("s")
        gwid = c * NS + s

        def zrow(i, carry):
            for v in range(D // 16):
                g0[i, pl.ds(16 * v, 16)] = jnp.zeros((16,), jnp.float32)
            return carry
        lax.fori_loop(0, ZROWS, zrow, 0)
        base = s * ROWS_PER_TILE
        for t in range(ROWS_PER_TILE // ZROWS):
            pltpu.sync_copy(g0, acc.at[pl.ds(base + t * ZROWS, ZROWS)])
        plsc.subcore_barrier()

        def scale(j, src_buf, dst_buf):
            def group(g, gcarry):
                wv = ew_v[j, pl.ds(g * 16, 16)]
                ws = [jnp.full((16,), wv[i], jnp.float32) for i in range(16)]

                def col(v, ccarry):
                    sl = pl.ds(v * 16, 16)
                    for i in range(16):
                        e = g * 16 + i
                        dst_buf[e, sl] = src_buf[e, sl] * ws[i]
                    return ccarry
                lax.fori_loop(0, D // 16, col, 0)
                return gcarry
            lax.fori_loop(0, CHUNK // 16, group, 0)

        for phase in range(N_PHASES):
            p0 = phase * PH_CHUNKS
            pltpu.sync_copy(src_hbm.at[gwid, pl.ds(p0, PH_CHUNKS)], src_v)
            pltpu.sync_copy(dst_hbm.at[gwid, pl.ds(p0, PH_CHUNKS)], dst_v)
            pltpu.sync_copy(ew_hbm.at[gwid, pl.ds(p0, PH_CHUNKS)], ew_v)

            for b in range(2):
                pltpu.async_copy(x_hbm.at[src_v.at[b]], gbufs[b], sems_g[b])

            def pair(q, carry):
                for b in range(2):
                    j = 2 * q + b
                    jn = jnp.minimum(j + 2, PH_CHUNKS - 1)

                    pltpu.make_async_copy(
                        x_hbm.at[src_v.at[j]], gbufs[b], sems_g[b]).wait()

                    pltpu.async_copy(x_hbm.at[src_v.at[jn]], gbufs[b],
                                     sems_g[b])
                return carry
            lax.fori_loop(0, PH_CHUNKS // 2, pair, 0)
            for b in range(2):
                pltpu.make_async_copy(
                    x_hbm.at[src_v.at[0]], gbufs[b], sems_g[b]).wait()

        plsc.subcore_barrier()
        for t in range(ROWS_PER_TILE // ZROWS):
            lo = base + t * ZROWS
            pltpu.sync_copy(acc.at[pl.ds(lo, ZROWS)],
                            out_hbm.at[c, pl.ds(lo, ZROWS)])

    return k(X, src, dst, ew)


def _tc_body(p0_ref, p1_ref, w_ref, b_ref, o_ref):
    o_ref[...] = (
        lax.dot_general(p0_ref[...] + p1_ref[...], w_ref[...],
                        (((1,), (1,)), ((), ())),
                        preferred_element_type=jnp.float32)
        + b_ref[...])


def _tc_linear(p0, p1, W, b2d):
    rows = 1000
    return pl.pallas_call(
        _tc_body,
        grid=(N_NODES // rows,),
        in_specs=[
            pl.BlockSpec((rows, D), lambda i: (i, 0)),
            pl.BlockSpec((rows, D), lambda i: (i, 0)),
            pl.BlockSpec((D, D), lambda i: (0, 0)),
            pl.BlockSpec((1, D), lambda i: (0, 0)),
        ],
        out_specs=pl.BlockSpec((rows, D), lambda i: (i, 0)),
        out_shape=jax.ShapeDtypeStruct((N_NODES, D), jnp.float32),
    )(p0, p1, W, b2d)


def kernel(X, edge_index, edge_weight, W, b):
    src = edge_index[1].astype(jnp.int32)
    dst = edge_index[0].astype(jnp.int32)
    ew = edge_weight.astype(jnp.float32)
    pad = E_PAD - src.shape[0]
    src = jnp.pad(src, (0, pad)).reshape(NW, N_CHUNKS, CHUNK)
    dst = jnp.pad(dst, (0, pad)).reshape(NW, N_CHUNKS, CHUNK)
    ew = jnp.pad(ew, (0, pad)).reshape(NW, N_CHUNKS, CHUNK)
    part = _sc_scatter(X, src, dst, ew)
    return _tc_linear(part[0, :N_NODES], part[1, :N_NODES], W,
                      b.reshape(1, D))


# speedup vs baseline: 1.4898x; 1.2287x over previous
import functools

import jax
import jax.numpy as jnp
from jax import lax
from jax.experimental import pallas as pl
from jax.experimental.pallas import tpu as pltpu
from jax.experimental.pallas import tpu_sc as plsc

N_NODES = 10000
D = 128
DH = D // 2
DP = DH // 2
NC = 2
NS = 16
CHUNK = 128
N_CHUNKS = 160
N_PHASES = 2
PH_CHUNKS = N_CHUNKS // N_PHASES
E_PAD = NS * N_CHUNKS * CHUNK
N_ACC = 10240
ROWS_PER_TILE = N_ACC // NS
ZROWS = 128


def _sc_scatter(T, src, dst, ew):
    mesh = plsc.VectorSubcoreMesh(
        core_axis_name="c", subcore_axis_name="s",
        num_cores=NC, num_subcores=NS)

    @functools.partial(
        pl.kernel,
        out_type=jax.ShapeDtypeStruct((NC, N_ACC, DH), jnp.float32),
        mesh=mesh,
        scratch_types=[
            pltpu.VMEM((PH_CHUNKS, CHUNK), jnp.int32),
            pltpu.VMEM((PH_CHUNKS, CHUNK), jnp.int32),
            pltpu.VMEM((PH_CHUNKS, CHUNK), jnp.float32),
            pltpu.VMEM((CHUNK, DP), jnp.uint32),
            pltpu.VMEM((CHUNK, DP), jnp.uint32),
            pltpu.VMEM((CHUNK, DP), jnp.uint32),
            pltpu.VMEM((CHUNK, DP), jnp.uint32),
            pltpu.VMEM((CHUNK, DH), jnp.float32),
            pltpu.VMEM((CHUNK, DH), jnp.float32),
            pltpu.VMEM_SHARED((N_ACC, DH), jnp.float32),
            pltpu.SemaphoreType.DMA,
            pltpu.SemaphoreType.DMA,
            pltpu.SemaphoreType.DMA,
            pltpu.SemaphoreType.DMA,
            pltpu.SemaphoreType.DMA,
            pltpu.SemaphoreType.DMA,
        ],
        compiler_params=pltpu.CompilerParams(use_tc_tiling_on_sc=False,
                                             needs_layout_passes=False),
    )
    def k(t_hbm, src_hbm, dst_hbm, ew_hbm, out_hbm,
          src_v, dst_v, ew_v, g0, g1, g2, g3, s0, s1, acc,
          sem_g0, sem_g1, sem_g2, sem_g3, sem_s0, sem_s1):
        gbufs = (g0, g1, g2, g3)
        sbufs = (s0, s1)
        sems_g = (sem_g0, sem_g1, sem_g2, sem_g3)
        sems_s = (sem_s0, sem_s1)
        c = lax.axis_index("c")
        s = lax.axis_index("s")

        def zrow(i, carry):
            for v in range(DH // 16):
                s0[i, pl.ds(16 * v, 16)] = jnp.zeros((16,), jnp.float32)
            return carry
        lax.fori_loop(0, ZROWS, zrow, 0)
        base = s * ROWS_PER_TILE
        for t in range(ROWS_PER_TILE // ZROWS):
            pltpu.sync_copy(s0, acc.at[pl.ds(base + t * ZROWS, ZROWS)])
        plsc.subcore_barrier()

        row0 = c * N_NODES
        himask = jnp.full((16,), 0xFFFF0000, jnp.uint32)

        def scale(j, src_buf, dst_buf):
            def group(g, gcarry):
                wv = ew_v[j, pl.ds(g * 16, 16)]
                ws = [jnp.full((16,), wv[i], jnp.float32) for i in range(16)]

                def blk(v, bcarry):
                    psl = pl.ds(v * 16, 16)
                    for i in range(16):
                        e = g * 16 + i
                        xi = src_buf[e, psl]
                        lo = plsc.bitcast(xi << 16, jnp.float32)
                        hi = plsc.bitcast(xi & himask, jnp.float32)
                        dst_buf[e, pl.ds(v * 32, 16)] = lo * ws[i]
                        dst_buf[e, pl.ds(v * 32 + 16, 16)] = hi * ws[i]
                    return bcarry
                lax.fori_loop(0, DP // 16, blk, 0)
                return gcarry
            lax.fori_loop(0, CHUNK // 16, group, 0)

        for phase in range(N_PHASES):
            p0 = phase * PH_CHUNKS
            pltpu.sync_copy(src_hbm.at[s, pl.ds(p0, PH_CHUNKS)], src_v)
            pltpu.sync_copy(dst_hbm.at[s, pl.ds(p0, PH_CHUNKS)], dst_v)
            pltpu.sync_copy(ew_hbm.at[s, pl.ds(p0, PH_CHUNKS)], ew_v)

            def rebase(i, carry):
                for v in range(CHUNK // 16):
                    sl = pl.ds(16 * v, 16)
                    src_v[i, sl] = src_v[i, sl] + row0
                return carry
            lax.fori_loop(0, PH_CHUNKS, rebase, 0)

            for b in range(4):
                pltpu.async_copy(t_hbm.at[src_v.at[b]], gbufs[b], sems_g[b])

            def quad(q, carry):
                for b in range(4):
                    j = 4 * q + b
                    sb = b % 2
                    jn = jnp.minimum(j + 4, PH_CHUNKS - 1)

                    pltpu.make_async_copy(
                        t_hbm.at[src_v.at[j]], gbufs[b], sems_g[b]).wait()

                    @pl.when(j >= 2)
                    def _():
                        pltpu.make_async_copy(
                            sbufs[sb], acc.at[dst_v.at[j]], sems_s[sb]).wait()
                    scale(j, gbufs[b], sbufs[sb])
                    pltpu.async_copy(sbufs[sb], acc.at[dst_v.at[j]],
                                     sems_s[sb], add=True)
                    pltpu.async_copy(t_hbm.at[src_v.at[jn]], gbufs[b],
                                     sems_g[b])
                return carry
            lax.fori_loop(0, PH_CHUNKS // 4, quad, 0)
            for b in range(4):
                pltpu.make_async_copy(
                    t_hbm.at[src_v.at[0]], gbufs[b], sems_g[b]).wait()
            for sb in range(2):
                pltpu.make_async_copy(
                    sbufs[sb], acc.at[dst_v.at[0]], sems_s[sb]).wait()

        plsc.subcore_barrier()
        for t in range(ROWS_PER_TILE // ZROWS):
            lo = base + t * ZROWS
            pltpu.sync_copy(acc.at[pl.ds(lo, ZROWS)],
                            out_hbm.at[c, pl.ds(lo, ZROWS)])

    return k(T, src, dst, ew)


def _pack_table(X):
    halves = jnp.concatenate([X[:, :DH], X[:, DH:]], axis=0)
    b16 = halves.astype(jnp.bfloat16)
    u16 = jax.lax.bitcast_convert_type(b16, jnp.uint16).astype(jnp.uint32)
    grp = u16.reshape(NC * N_NODES, DH // 32, 2, 16)
    packed = grp[:, :, 0, :] | (grp[:, :, 1, :] << 16)
    return packed.reshape(NC * N_NODES, DP)


def _tc_body(p0_ref, p1_ref, w0_ref, w1_ref, b_ref, o_ref):
    o_ref[...] = (
        lax.dot_general(p0_ref[...], w0_ref[...], (((1,), (1,)), ((), ())),
                        preferred_element_type=jnp.float32)
        + lax.dot_general(p1_ref[...], w1_ref[...], (((1,), (1,)), ((), ())),
                          preferred_element_type=jnp.float32)
        + b_ref[...])


def _tc_linear(p0, p1, w0, w1, b2d):
    rows = 1000
    return pl.pallas_call(
        _tc_body,
        grid=(N_NODES // rows,),
        in_specs=[
            pl.BlockSpec((rows, DH), lambda i: (i, 0)),
            pl.BlockSpec((rows, DH), lambda i: (i, 0)),
            pl.BlockSpec((D, DH), lambda i: (0, 0)),
            pl.BlockSpec((D, DH), lambda i: (0, 0)),
            pl.BlockSpec((1, D), lambda i: (0, 0)),
        ],
        out_specs=pl.BlockSpec((rows, D), lambda i: (i, 0)),
        out_shape=jax.ShapeDtypeStruct((N_NODES, D), jnp.float32),
    )(p0, p1, w0, w1, b2d)


def kernel(X, edge_index, edge_weight, W, b):
    src = edge_index[1].astype(jnp.int32)
    dst = edge_index[0].astype(jnp.int32)
    ew = edge_weight.astype(jnp.float32)
    pad = E_PAD - src.shape[0]
    src = jnp.pad(src, (0, pad)).reshape(NS, N_CHUNKS, CHUNK)
    dst = jnp.pad(dst, (0, pad)).reshape(NS, N_CHUNKS, CHUNK)
    ew = jnp.pad(ew, (0, pad)).reshape(NS, N_CHUNKS, CHUNK)
    part = _sc_scatter(_pack_table(X), src, dst, ew)
    return _tc_linear(part[0, :N_NODES], part[1, :N_NODES],
                      W[:, :DH], W[:, DH:], b.reshape(1, D))
